# Initial kernel scaffold; baseline (speedup 1.0000x reference)
import jax
import jax.numpy as jnp
from jax.experimental import pallas as pl


def kernel(atomic_numbers, pair_indices, f_ij, emb_table, W_rbf, b_rbf, W_out, b_out):
    raise NotImplementedError("write your pallas kernel here")


# speedup vs baseline: 6.0006x; 6.0006x over previous
import functools

import jax
import jax.numpy as jnp
from jax import lax
from jax.experimental import pallas as pl
from jax.experimental.pallas import tpu as pltpu
from jax.experimental.pallas import tpu_sc as plsc

N_NODES = 10000
N_EDGES = 320000
EMB = 128
N_RBF = 6
NUM_EMB = 95

_NC = 2
_NS = 16
_NW = _NC * _NS
_EPW = N_EDGES // _NW
_CH = 80
_NCHUNK = _EPW // _CH


def _silu(x):
    return x * jax.nn.sigmoid(x)


def _tables_body(an_ref, emb_ref, w1_ref, w2_ref, bout_ref, c1_ref, c2_ref):
    an = an_ref[...]
    classes = lax.broadcasted_iota(jnp.int32, (an.shape[0], 128), 1)
    oh = (an == classes).astype(jnp.float32)
    e = jnp.dot(oh, emb_ref[...], preferred_element_type=jnp.float32)
    c1_ref[...] = (
        jnp.dot(e, w1_ref[...], preferred_element_type=jnp.float32)
        + bout_ref[...]
    )
    c2_ref[...] = jnp.dot(e, w2_ref[...], preferred_element_type=jnp.float32)


def _build_tables(an2, emb_pad, w1, w2, bout2):
    nb = 1000
    grid = N_NODES // nb
    return pl.pallas_call(
        _tables_body,
        grid=(grid,),
        in_specs=[
            pl.BlockSpec((nb, 1), lambda i: (i, 0)),
            pl.BlockSpec((128, EMB), lambda i: (0, 0)),
            pl.BlockSpec((EMB, EMB), lambda i: (0, 0)),
            pl.BlockSpec((EMB, EMB), lambda i: (0, 0)),
            pl.BlockSpec((1, EMB), lambda i: (0, 0)),
        ],
        out_specs=[
            pl.BlockSpec((nb, EMB), lambda i: (i, 0)),
            pl.BlockSpec((nb, EMB), lambda i: (i, 0)),
        ],
        out_shape=[
            jax.ShapeDtypeStruct((N_NODES, EMB), jnp.float32),
            jax.ShapeDtypeStruct((N_NODES, EMB), jnp.float32),
        ],
    )(an2, emb_pad, w1, w2, bout2)


def _sc_gather(c1, c2, pi, pj):
    mesh = plsc.VectorSubcoreMesh(core_axis_name="c", subcore_axis_name="s")

    @functools.partial(
        pl.kernel,
        mesh=mesh,
        out_type=[
            jax.ShapeDtypeStruct((N_EDGES, EMB), jnp.float32),
            jax.ShapeDtypeStruct((N_EDGES, EMB), jnp.float32),
        ],
        scratch_types=[
            pltpu.VMEM((_CH,), jnp.int32),
            pltpu.VMEM((_CH,), jnp.int32),
            pltpu.VMEM((_CH, EMB), jnp.float32),
            pltpu.VMEM((_CH, EMB), jnp.float32),
            pltpu.SemaphoreType.DMA,
            pltpu.SemaphoreType.DMA,
        ],
    )
    def k(c1_hbm, c2_hbm, pi_hbm, pj_hbm, g1_hbm, g2_hbm,
          idx1, idx2, rows1, rows2, sem1, sem2):
        wid = lax.axis_index("s") * _NC + lax.axis_index("c")

        def body(c, carry):
            base = wid * _EPW + c * _CH
            pltpu.sync_copy(pi_hbm.at[pl.ds(base, _CH)], idx1)
            pltpu.sync_copy(pj_hbm.at[pl.ds(base, _CH)], idx2)
            cp1 = pltpu.async_copy(c1_hbm.at[idx1], rows1, sem1)
            cp2 = pltpu.async_copy(c2_hbm.at[idx2], rows2, sem2)
            cp1.wait()
            cp2.wait()
            pltpu.sync_copy(rows1, g1_hbm.at[pl.ds(base, _CH)])
            pltpu.sync_copy(rows2, g2_hbm.at[pl.ds(base, _CH)])
            return carry

        lax.fori_loop(0, _NCHUNK, body, 0)

    return k(c1, c2, pi, pj)


def _main_body(g1_ref, g2_ref, f_ref, wr_ref, br_ref, w3_ref, out_ref):
    r = jnp.dot(f_ref[...], wr_ref[...], preferred_element_type=jnp.float32)
    r = _silu(_silu(r + br_ref[...]))
    d = jnp.dot(r, w3_ref[...], preferred_element_type=jnp.float32)
    out_ref[...] = _silu(d + g1_ref[...] + g2_ref[...])


def _main_call(g1, g2, f_pad, wr_pad, br2, w3):
    eb = 4000
    grid = N_EDGES // eb
    return pl.pallas_call(
        _main_body,
        grid=(grid,),
        in_specs=[
            pl.BlockSpec((eb, EMB), lambda i: (i, 0)),
            pl.BlockSpec((eb, EMB), lambda i: (i, 0)),
            pl.BlockSpec((eb, 8), lambda i: (i, 0)),
            pl.BlockSpec((8, EMB), lambda i: (0, 0)),
            pl.BlockSpec((1, EMB), lambda i: (0, 0)),
            pl.BlockSpec((EMB, EMB), lambda i: (0, 0)),
        ],
        out_specs=pl.BlockSpec((eb, EMB), lambda i: (i, 0)),
        out_shape=jax.ShapeDtypeStruct((N_EDGES, EMB), jnp.float32),
    )(g1, g2, f_pad, wr_pad, br2, w3)


def kernel(atomic_numbers, pair_indices, f_ij, emb_table, W_rbf, b_rbf,
           W_out, b_out):
    an2 = atomic_numbers.astype(jnp.int32).reshape(N_NODES, 1)
    emb_pad = jnp.zeros((128, EMB), jnp.float32).at[:NUM_EMB].set(emb_table)
    w1 = W_out[:EMB]
    w2 = W_out[EMB:2 * EMB]
    w3 = W_out[2 * EMB:]
    c1, c2 = _build_tables(an2, emb_pad, w1, w2, b_out.reshape(1, EMB))

    pi = pair_indices[0].astype(jnp.int32)
    pj = pair_indices[1].astype(jnp.int32)
    g1, g2 = _sc_gather(c1, c2, pi, pj)

    f_pad = jnp.pad(f_ij, ((0, 0), (0, 2)))
    wr_pad = jnp.pad(W_rbf, ((0, 2), (0, 0)))
    return _main_call(g1, g2, f_pad, wr_pad, b_rbf.reshape(1, EMB), w3)
